# baseline (device time: 10120 ns/iter reference)
import jax
import jax.numpy as jnp
from jax import lax
from jax.experimental import pallas as pl
from jax.experimental.pallas import tpu as pltpu

N_GLOBAL = 512
EPS = 1e-5


def kernel(x, gamma, beta):
    m, n = x.shape

    def body(x_ref, g_ref, b_ref, out_ref, stats_ref, recv_ref, send_sem, recv_sem):
        my_x = lax.axis_index("x")
        my_y = lax.axis_index("y")
        peer = (my_x, 1 - my_y)

        barrier_sem = pltpu.get_barrier_semaphore()
        pl.semaphore_signal(
            barrier_sem, inc=1, device_id=peer, device_id_type=pl.DeviceIdType.MESH
        )
        pl.semaphore_wait(barrier_sem, 1)

        xv = x_ref[...]
        s = jnp.sum(xv, axis=1, keepdims=True)
        ss = jnp.sum(xv * xv, axis=1, keepdims=True)
        stats_ref[...] = jnp.concatenate([s, ss], axis=1)

        rdma = pltpu.make_async_remote_copy(
            src_ref=stats_ref,
            dst_ref=recv_ref,
            send_sem=send_sem,
            recv_sem=recv_sem,
            device_id=peer,
            device_id_type=pl.DeviceIdType.MESH,
        )
        rdma.start()
        rdma.wait()

        g = g_ref[...].reshape(1, xv.shape[1])
        b = b_ref[...].reshape(1, xv.shape[1])
        tot = stats_ref[...] + recv_ref[...]
        mean = tot[:, 0:1] * (1.0 / N_GLOBAL)
        var = tot[:, 1:2] * (1.0 / N_GLOBAL) - mean * mean
        inv = lax.rsqrt(var + EPS)
        out_ref[...] = g * ((xv - mean) * inv) + b

    return pl.pallas_call(
        body,
        out_shape=jax.ShapeDtypeStruct((m, n), jnp.float32),
        in_specs=[
            pl.BlockSpec(memory_space=pltpu.VMEM),
            pl.BlockSpec(memory_space=pltpu.VMEM),
            pl.BlockSpec(memory_space=pltpu.VMEM),
        ],
        out_specs=pl.BlockSpec(memory_space=pltpu.VMEM),
        scratch_shapes=[
            pltpu.VMEM((m, 2), jnp.float32),
            pltpu.VMEM((m, 2), jnp.float32),
            pltpu.SemaphoreType.DMA,
            pltpu.SemaphoreType.DMA,
        ],
        compiler_params=pltpu.CompilerParams(collective_id=0),
    )(x, gamma, beta)


# device time: 7602 ns/iter; 1.3312x vs baseline; 1.3312x over previous
import jax
import jax.numpy as jnp
from jax import lax
from jax.experimental import pallas as pl
from jax.experimental.pallas import tpu as pltpu

N_GLOBAL = 512
EPS = 1e-5


def kernel(x, gamma, beta):
    m, n = x.shape

    def body(x_ref, g_ref, b_ref, out_ref, stats_ref, recv_ref, send_sem, recv_sem):
        my_x = lax.axis_index("x")
        my_y = lax.axis_index("y")
        peer = (my_x, 1 - my_y)

        barrier_sem = pltpu.get_barrier_semaphore()
        pl.semaphore_signal(
            barrier_sem, inc=1, device_id=peer, device_id_type=pl.DeviceIdType.MESH
        )
        pl.semaphore_wait(barrier_sem, 1)

        xv = x_ref[...]
        ones = jnp.ones((1, n), jnp.float32)
        dn = (((1,), (1,)), ((), ()))
        s_row = lax.dot_general(ones, xv, dn, preferred_element_type=jnp.float32)
        ss_row = lax.dot_general(ones, xv * xv, dn, preferred_element_type=jnp.float32)
        local = jnp.concatenate([s_row, ss_row], axis=0)
        stats_ref[...] = local

        rdma = pltpu.make_async_remote_copy(
            src_ref=stats_ref,
            dst_ref=recv_ref,
            send_sem=send_sem,
            recv_sem=recv_sem,
            device_id=peer,
            device_id_type=pl.DeviceIdType.MESH,
        )
        rdma.start()

        g = g_ref[...].reshape(1, n)
        b = b_ref[...].reshape(1, n)
        gx = xv * g

        rdma.wait_recv()
        tot = jnp.transpose(local + recv_ref[...])
        mean = tot[:, 0:1] * (1.0 / N_GLOBAL)
        var = tot[:, 1:2] * (1.0 / N_GLOBAL) - mean * mean
        inv = lax.rsqrt(var + EPS)
        out_ref[...] = gx * inv + (b - g * (mean * inv))
        rdma.wait_send()

    return pl.pallas_call(
        body,
        out_shape=jax.ShapeDtypeStruct((m, n), jnp.float32),
        in_specs=[
            pl.BlockSpec(memory_space=pltpu.VMEM),
            pl.BlockSpec(memory_space=pltpu.VMEM),
            pl.BlockSpec(memory_space=pltpu.VMEM),
        ],
        out_specs=pl.BlockSpec(memory_space=pltpu.VMEM),
        scratch_shapes=[
            pltpu.VMEM((2, m), jnp.float32),
            pltpu.VMEM((2, m), jnp.float32),
            pltpu.SemaphoreType.DMA,
            pltpu.SemaphoreType.DMA,
        ],
        compiler_params=pltpu.CompilerParams(collective_id=0),
    )(x, gamma, beta)


# device time: 6245 ns/iter; 1.6205x vs baseline; 1.2173x over previous
import jax
import jax.numpy as jnp
from jax import lax
from jax.experimental import pallas as pl
from jax.experimental.pallas import tpu as pltpu

N_GLOBAL = 512
EPS = 1e-5


def kernel(x, gamma, beta):
    m, n = x.shape

    def body(x_ref, g_ref, b_ref, out_ref, stats_ref, recv_ref, send_sem, recv_sem):
        my_x = lax.axis_index("x")
        my_y = lax.axis_index("y")
        peer = (my_x, 1 - my_y)

        xv = x_ref[...]
        ones = jnp.ones((1, n), jnp.float32)
        dn = (((1,), (1,)), ((), ()))
        s_row = lax.dot_general(ones, xv, dn, preferred_element_type=jnp.float32)
        ss_row = lax.dot_general(ones, xv * xv, dn, preferred_element_type=jnp.float32)
        local = jnp.concatenate([s_row, ss_row], axis=0)
        stats_ref[...] = local

        barrier_sem = pltpu.get_barrier_semaphore()
        pl.semaphore_signal(
            barrier_sem, inc=1, device_id=peer, device_id_type=pl.DeviceIdType.MESH
        )
        pl.semaphore_wait(barrier_sem, 1)

        rdma = pltpu.make_async_remote_copy(
            src_ref=stats_ref,
            dst_ref=recv_ref,
            send_sem=send_sem,
            recv_sem=recv_sem,
            device_id=peer,
            device_id_type=pl.DeviceIdType.MESH,
        )
        rdma.start()

        g = g_ref[...].reshape(1, n)
        b = b_ref[...].reshape(1, n)
        gx = xv * g

        rdma.wait_recv()
        tot = jnp.transpose(local + recv_ref[...])
        mean = tot[:, 0:1] * (1.0 / N_GLOBAL)
        var = tot[:, 1:2] * (1.0 / N_GLOBAL) - mean * mean
        inv = lax.rsqrt(var + EPS)
        out_ref[...] = gx * inv + (b - g * (mean * inv))
        rdma.wait_send()

    return pl.pallas_call(
        body,
        out_shape=jax.ShapeDtypeStruct((m, n), jnp.float32),
        in_specs=[
            pl.BlockSpec(memory_space=pltpu.VMEM),
            pl.BlockSpec(memory_space=pltpu.VMEM),
            pl.BlockSpec(memory_space=pltpu.VMEM),
        ],
        out_specs=pl.BlockSpec(memory_space=pltpu.VMEM),
        scratch_shapes=[
            pltpu.VMEM((2, m), jnp.float32),
            pltpu.VMEM((2, m), jnp.float32),
            pltpu.SemaphoreType.DMA,
            pltpu.SemaphoreType.DMA,
        ],
        input_output_aliases={0: 0},
        compiler_params=pltpu.CompilerParams(collective_id=0),
    )(x, gamma, beta)
